# Initial kernel scaffold; baseline (speedup 1.0000x reference)
import jax
import jax.numpy as jnp
from jax.experimental import pallas as pl


def kernel(x, y_hat, temperature, classifier_w, basic_state, factory_bias, mix_head_w, mix_head_b, mix_w_real, mix_w_imag, mix_b_real, mix_b_imag):
    raise NotImplementedError("write your pallas kernel here")


# speedup vs baseline: 58.4482x; 58.4482x over previous
import functools
import math

import jax
import jax.numpy as jnp
import numpy as np
from jax.experimental import pallas as pl
from jax.experimental.pallas import tpu as pltpu

SEQ_LEN = 512
PRED_LEN = 96
C = 64
K = 8
STATE_NUM = 8
B = 16
F_DIM = PRED_LEN // 2 + 1
OUT_DIM = F_DIM * (2 * K + 1)
IC = 16
N_IC = C // IC
SEQT = SEQ_LEN + PRED_LEN
NLAG = SEQ_LEN - 2


def _dft_constants():
    s = np.arange(SEQ_LEN)[:, None]
    f = np.arange(SEQ_LEN // 2 + 1)[None, :]
    ang = 2.0 * np.pi * s * f / SEQ_LEN
    fre = np.cos(ang)
    fim = -np.sin(ang)
    w = np.full((SEQ_LEN // 2 + 1,), 2.0)
    w[0] = 1.0
    w[-1] = 1.0
    t = np.arange(SEQ_LEN)[None, :]
    fa = np.arange(SEQ_LEN // 2 + 1)[:, None]
    ang2 = 2.0 * np.pi * fa * t / SEQ_LEN
    inv_a = w[:, None] * np.cos(ang2) / SEQ_LEN
    inv_b = -w[:, None] * np.sin(ang2) / SEQ_LEN
    s9 = np.arange(PRED_LEN)[:, None]
    f9 = np.arange(F_DIM)[None, :]
    ang9 = 2.0 * np.pi * s9 * f9 / PRED_LEN
    f96re = np.cos(ang9)
    f96im = -np.sin(ang9)
    w9 = np.full((F_DIM,), 2.0)
    w9[0] = 1.0
    w9[-1] = 1.0
    fg = np.arange(F_DIM)[:, None]
    tg = np.arange(PRED_LEN)[None, :]
    angg = 2.0 * np.pi * fg * tg / PRED_LEN
    g_re = w9[:, None] * np.cos(angg) / PRED_LEN
    g_im = -w9[:, None] * np.sin(angg) / PRED_LEN
    c = lambda a: np.asarray(a, np.float32)
    return c(fre), c(fim), c(inv_a), c(inv_b), c(f96re), c(f96im), c(g_re), c(g_im)


_HI = jax.lax.Precision.HIGHEST


def _dot(a, b):
    return jax.lax.dot_general(a, b, (((a.ndim - 1,), (0,)), ((), ())),
                               precision=_HI, preferred_element_type=jnp.float32)


def _lift_kernel(x_ref, y_ref, xi_ref, yi_ref, temp_ref, cwt_ref, bs_ref,
                 fb_ref, mhw_ref, mhb_ref, m1_ref, bc_ref, fre_ref, fim_ref,
                 inva_ref, invb_ref, f96re_ref, f96im_ref, gre_ref, gim_ref,
                 out_ref):
    x = x_ref[0]
    yh = y_ref[0]
    x_i = xi_ref[0]
    yh_i = yi_ref[0]

    mu = jnp.mean(x, axis=-1, keepdims=True)
    xc = x - mu
    std = jnp.sqrt(jnp.mean(xc * xc, axis=-1, keepdims=True) + 1e-8)
    xn = xc / std
    yn = (yh - mu) / std
    mu_i = jnp.mean(x_i, axis=-1, keepdims=True)
    xc_i = x_i - mu_i
    std_i = jnp.sqrt(jnp.mean(xc_i * xc_i, axis=-1, keepdims=True) + 1e-8)
    xn_i = xc_i / std_i
    yn_i = (yh_i - mu_i) / std_i

    rr = _dot(xn, fre_ref[...])
    ri = _dot(xn, fim_ref[...])
    rr_i = _dot(xn_i, fre_ref[...])
    ri_i = _dot(xn_i, fim_ref[...])

    p_re = rr_i[:, None, :] * rr[None, :, :] + ri_i[:, None, :] * ri[None, :, :]
    p_im = ri_i[:, None, :] * rr[None, :, :] - rr_i[:, None, :] * ri[None, :, :]
    p_re = p_re.reshape(IC * C, SEQ_LEN // 2 + 1)
    p_im = p_im.reshape(IC * C, SEQ_LEN // 2 + 1)
    cc = _dot(p_re, inva_ref[...]) + _dot(p_im, invb_ref[...])
    cc = cc.reshape(IC, C, SEQ_LEN)

    ca = jnp.abs(cc)
    m = (ca[:, :, 1:-1] >= ca[:, :, :-2]) & (ca[:, :, 1:-1] >= ca[:, :, 2:])
    ccm = jnp.where(m, cc[:, :, 1:-1], 0.0) * (1.0 / SEQ_LEN)
    cam = jnp.abs(ccm)
    camax = jnp.max(cam, axis=-1)
    lag_iota = jax.lax.broadcasted_iota(jnp.int32, (IC, C, NLAG), 2)
    hit = cam >= camax[:, :, None]
    sh_rel = jnp.min(jnp.where(hit, lag_iota, NLAG), axis=-1)
    first = lag_iota == sh_rel[:, :, None]
    r_val = jnp.sum(jnp.where(first, ccm, 0.0), axis=-1)
    shift = sh_rel + 1

    j_iota = jax.lax.broadcasted_iota(jnp.int32, (IC, C), 1)
    cur = camax
    lead_l, shift_l, r_l = [], [], []
    for _ in range(K):
        mx = jnp.max(cur, axis=-1, keepdims=True)
        idx = jnp.min(jnp.where(cur >= mx, j_iota, C), axis=-1, keepdims=True)
        sel = j_iota == idx
        lead_l.append(idx)
        shift_l.append(jnp.sum(jnp.where(sel, shift, 0), axis=-1, keepdims=True))
        r_l.append(jnp.sum(jnp.where(sel, r_val, 0.0), axis=-1, keepdims=True))
        cur = jnp.where(sel, -1.0, cur)

    seq = jnp.concatenate([xn, yn], axis=-1)
    col_iota = jax.lax.broadcasted_iota(jnp.int32, (IC, C), 1)
    onehot = jnp.concatenate(
        [(lead_l[k] == col_iota).astype(jnp.float32) for k in range(K)], axis=0)
    rows = _dot(onehot, seq)
    shifts_km = jnp.concatenate(shift_l, axis=0)
    r_km = jnp.concatenate(r_l, axis=0)
    rows = rows * jnp.sign(r_km)
    for bit in range(9):
        amt = 1 << bit
        rolled = pltpu.roll(rows, amt, 1)
        rows = jnp.where((shifts_km & amt) != 0, rolled, rows)
    win = rows[:, SEQ_LEN:]

    t_inv = 1.0 / temp_ref[0, 0]
    r_abs = jnp.abs(jnp.concatenate(r_l, axis=1))
    z = jnp.concatenate([jnp.ones((IC, 1), jnp.float32), r_abs], axis=1) * t_inv
    z = z - jnp.max(z, axis=-1, keepdims=True)
    ez = jnp.exp(z)
    sm = ez / jnp.sum(ez, axis=-1, keepdims=True)
    cf = sm[:, 1:]

    logits = fb_ref[...] + bs_ref[...] + _dot(x_i, cwt_ref[...])
    logits = logits - jnp.max(logits, axis=-1, keepdims=True)
    el = jnp.exp(logits)
    p = el / jnp.sum(el, axis=-1, keepdims=True)

    filt = _dot(p, mhb_ref[...])
    for k in range(K):
        wk = mhw_ref[:, k * OUT_DIM:(k + 1) * OUT_DIM]
        filt = filt + cf[:, k:k + 1] * _dot(p, wk)

    yf_re = _dot(yn_i, f96re_ref[...])
    yf_im = _dot(yn_i, f96im_ref[...])
    ss_re = jnp.zeros((IC, F_DIM), jnp.float32)
    ss_im = jnp.zeros((IC, F_DIM), jnp.float32)
    sd_re = jnp.zeros((IC, F_DIM), jnp.float32)
    sd_im = jnp.zeros((IC, F_DIM), jnp.float32)
    for k in range(K):
        wk = win[k * IC:(k + 1) * IC, :]
        sf_re = _dot(wk, f96re_ref[...])
        sf_im = _dot(wk, f96im_ref[...])
        f1 = filt[:, k * F_DIM:(k + 1) * F_DIM]
        f2 = filt[:, (K + k) * F_DIM:(K + k + 1) * F_DIM]
        a_re = sf_re * f1
        a_im = sf_im * f1
        ss_re = ss_re + a_re
        ss_im = ss_im + a_im
        sd_re = sd_re + (a_re - yf_re) * f2
        sd_im = sd_im + (a_im - yf_im) * f2
    f_last = filt[:, 2 * K * F_DIM:]
    y2_re = yf_re * f_last
    y2_im = yf_im * f_last
    mix_re = jnp.concatenate([ss_re, sd_re, y2_re], axis=1)
    mix_im = jnp.concatenate([ss_im, sd_im, y2_im], axis=1)
    out_cat = _dot(jnp.concatenate([mix_re, mix_im], axis=1), m1_ref[...])
    out_cat = out_cat + bc_ref[...]
    y_add = (_dot(out_cat[:, :F_DIM], gre_ref[...]) +
             _dot(out_cat[:, F_DIM:], gim_ref[...]))

    out_ref[0] = (yn_i + y_add) * std_i + mu_i


@jax.jit
def kernel(x, y_hat, temperature, classifier_w, basic_state, factory_bias,
           mix_head_w, mix_head_b, mix_w_real, mix_w_imag, mix_b_real,
           mix_b_imag):
    fre, fim, inva, invb, f96re, f96im, gre, gim = _dft_constants()
    wr_t = mix_w_real.T
    wi_t = mix_w_imag.T
    m1 = jnp.concatenate([jnp.concatenate([wr_t, wi_t], axis=1),
                          jnp.concatenate([-wi_t, wr_t], axis=1)], axis=0)
    bc = jnp.concatenate([mix_b_real, mix_b_imag])[None, :]
    temp2 = temperature.reshape(1, 1)
    fb2 = factory_bias[None, :]
    cwt = classifier_w.T

    full = lambda shape: pl.BlockSpec(shape, lambda b, ic: (0,) * len(shape))
    grid = (B, N_IC)
    return pl.pallas_call(
        _lift_kernel,
        grid=grid,
        in_specs=[
            pl.BlockSpec((1, C, SEQ_LEN), lambda b, ic: (b, 0, 0)),
            pl.BlockSpec((1, C, PRED_LEN), lambda b, ic: (b, 0, 0)),
            pl.BlockSpec((1, IC, SEQ_LEN), lambda b, ic: (b, ic, 0)),
            pl.BlockSpec((1, IC, PRED_LEN), lambda b, ic: (b, ic, 0)),
            full((1, 1)),
            full((SEQ_LEN, STATE_NUM)),
            pl.BlockSpec((IC, STATE_NUM), lambda b, ic: (ic, 0)),
            full((1, STATE_NUM)),
            full((STATE_NUM, K * OUT_DIM)),
            full((STATE_NUM, OUT_DIM)),
            full((2 * 3 * F_DIM, 2 * F_DIM)),
            full((1, 2 * F_DIM)),
            full((SEQ_LEN, SEQ_LEN // 2 + 1)),
            full((SEQ_LEN, SEQ_LEN // 2 + 1)),
            full((SEQ_LEN // 2 + 1, SEQ_LEN)),
            full((SEQ_LEN // 2 + 1, SEQ_LEN)),
            full((PRED_LEN, F_DIM)),
            full((PRED_LEN, F_DIM)),
            full((F_DIM, PRED_LEN)),
            full((F_DIM, PRED_LEN)),
        ],
        out_specs=pl.BlockSpec((1, IC, PRED_LEN), lambda b, ic: (b, ic, 0)),
        out_shape=jax.ShapeDtypeStruct((B, C, PRED_LEN), jnp.float32),
        compiler_params=pltpu.CompilerParams(
            dimension_semantics=("parallel", "parallel")),
    )(x, y_hat, x, y_hat, temp2, cwt, basic_state, fb2, mix_head_w, mix_head_b,
      m1, bc, fre, fim, inva, invb, f96re, f96im, gre, gim)
